# Initial kernel scaffold; baseline (speedup 1.0000x reference)
import jax
import jax.numpy as jnp
from jax.experimental import pallas as pl


def kernel(bins, target_depth_maps):
    raise NotImplementedError("write your pallas kernel here")


# speedup vs baseline: 1.3035x; 1.3035x over previous
import jax
import jax.numpy as jnp
from jax.experimental import pallas as pl
from jax.experimental.pallas import tpu as pltpu

_B = 4
_L = 4
_P = 128
_M = 192 * 256
_CHUNK = 4096
_NCHUNK = _M // _CHUNK
_BIG = 1e10


def _body(centers_ref, y_ref, out_ref, chamx_ref, chamy_ref, cnt_ref):
    i = pl.program_id(0)

    @pl.when(i == 0)
    def _init():
        chamx_ref[...] = jnp.full((_L, _B, _P), _BIG, jnp.float32)
        chamy_ref[...] = jnp.zeros((_L, _B), jnp.float32)
        cnt_ref[...] = jnp.zeros((1, _B), jnp.float32)

    y = y_ref[...]
    mask = y >= 0.001
    cnt_ref[...] += jnp.sum(mask, axis=1, dtype=jnp.float32)[None, :]
    for l in range(_L):
        x = centers_ref[l]
        d2 = (x[:, :, None] - y[:, None, :]) ** 2
        d2m = jnp.where(mask[:, None, :], d2, _BIG)
        chamx_ref[l] = jnp.minimum(chamx_ref[l], jnp.min(d2m, axis=2))
        min_p = jnp.min(d2, axis=1)
        chamy_ref[l] += jnp.sum(jnp.where(mask, min_p, 0.0), axis=1)

    @pl.when(i == _NCHUNK - 1)
    def _fin():
        chamx = jnp.mean(chamx_ref[...], axis=2)
        chamy = chamy_ref[...] / cnt_ref[0][None, :]
        loss = jnp.sum(chamx + chamy) / jnp.float32(_B)
        out_ref[...] = loss[None, None]


def kernel(bins, target_depth_maps):
    y = target_depth_maps.reshape(_B, _M)
    centers = 0.5 * (bins[:, :, 1:] + bins[:, :, :-1])
    out = pl.pallas_call(
        _body,
        grid=(_NCHUNK,),
        in_specs=[
            pl.BlockSpec((_L, _B, _P), lambda i: (0, 0, 0)),
            pl.BlockSpec((_B, _CHUNK), lambda i: (0, i)),
        ],
        out_specs=pl.BlockSpec((1, 1), lambda i: (0, 0)),
        out_shape=jax.ShapeDtypeStruct((1, 1), jnp.float32),
        scratch_shapes=[
            pltpu.VMEM((_L, _B, _P), jnp.float32),
            pltpu.VMEM((_L, _B), jnp.float32),
            pltpu.VMEM((1, _B), jnp.float32),
        ],
        compiler_params=pltpu.CompilerParams(
            dimension_semantics=("arbitrary",),
        ),
    )(centers, y)
    return out[0, 0]
